# Initial kernel scaffold; baseline (speedup 1.0000x reference)
import jax
import jax.numpy as jnp
from jax.experimental import pallas as pl


def kernel(x, edge_index, logits, W1l, b1, W1r, bn_gamma, bn_beta, W2l, b2, W2r):
    raise NotImplementedError("write your pallas kernel here")


# speedup vs baseline: 8.7867x; 8.7867x over previous
import functools

import jax
import jax.numpy as jnp
from jax import lax
from jax.experimental import pallas as pl
from jax.experimental.pallas import tpu as pltpu
from jax.experimental.pallas import tpu_sc as plsc

N = 50000
E = 800000
D = 100
H = 128
O = 2

NC = 2
NS = 16
LANES = 128
EB = E // LANES
CH = 4
CW = 32
DP = CH * CW
CNT_COL = 4
K2 = 16
R_MAIN = 3120
R_EX_TILES = (N - NS * R_MAIN) // 8
ZROWS = 208

SB = 4
G1 = 24

B_MAIN1 = 384
NG1 = B_MAIN1 // G1
EX8_1 = 13
EX8_OFF1 = NS * B_MAIN1
EX2_OFF = 6248

HB2 = 3128
B_MAIN2 = 192
NG2 = B_MAIN2 // G1
EX8_OFF2 = NS * B_MAIN2

BN = 1000
GQ = N // BN

def _zero_fill(zb, width):
    z16 = jnp.zeros((16,), jnp.float32)

    def body(i, _):
        for w in range(width // 16):
            zb[i, w * 16:(w + 1) * 16] = z16
        return 0

    lax.fori_loop(0, ZROWS, body, 0)


def _run_blocks(table, idx_src, dst_ref, agg_sh, si, di, msg, gsem, off, nblk):
    pltpu.sync_copy(idx_src(off, nblk), si.at[pl.ds(0, nblk)])
    pltpu.sync_copy(dst_ref.at[pl.ds(off, nblk)], di.at[pl.ds(0, nblk)])
    for w0 in range(0, nblk, SB):
        wn = min(SB, nblk - w0)
        descs = [
            pltpu.async_copy(table.at[si.at[w0 + j]], msg.at[j], gsem)
            for j in range(wn)
        ]
        for d in descs:
            d.wait()
        for j in range(wn):
            pltpu.sync_copy(msg.at[j], agg_sh.at[di.at[w0 + j]], add=True)


def _edge_pass(table, idx_src, dst_ref, agg_sh, si, di, msg, gsem,
               blk0, ngroups, ex8_pred, ex8_off, ex2_pred, ex2_off):
    def group(k, _):
        gb = blk0 + k * G1
        pltpu.sync_copy(idx_src(gb, G1), si)
        pltpu.sync_copy(dst_ref.at[pl.ds(gb, G1)], di)

        def sblock(s, _):
            descs = [
                pltpu.async_copy(table.at[si.at[s * SB + j]], msg.at[j], gsem)
                for j in range(SB)
            ]
            for d in descs:
                d.wait()
            for j in range(SB):
                pltpu.sync_copy(msg.at[j], agg_sh.at[di.at[s * SB + j]], add=True)
            return 0

        lax.fori_loop(0, G1 // SB, sblock, 0)
        return 0

    lax.fori_loop(0, ngroups, group, 0)

    @pl.when(ex8_pred)
    def _():
        _run_blocks(table, idx_src, dst_ref, agg_sh, si, di, msg, gsem,
                    ex8_off, 8)

    @pl.when(ex2_pred)
    def _():
        _run_blocks(table, idx_src, dst_ref, agg_sh, si, di, msg, gsem,
                    ex2_off, 2)


@functools.lru_cache(maxsize=1)
def _build_sc_kernels():
    mesh = plsc.VectorSubcoreMesh(
        core_axis_name="c", subcore_axis_name="s", num_cores=NC, num_subcores=NS)
    params = pltpu.CompilerParams(use_tc_tiling_on_sc=False)

    @functools.partial(
        pl.kernel,
        out_type=jax.ShapeDtypeStruct((CH, N, CW), jnp.float32),
        mesh=mesh,
        compiler_params=params,
        scratch_types=[
            pltpu.VMEM_SHARED((N, CW), jnp.float32),
            pltpu.VMEM((ZROWS, CW), jnp.float32),
            pltpu.VMEM((G1, LANES), jnp.int32),
            pltpu.VMEM((G1, LANES), jnp.int32),
            pltpu.VMEM((SB, LANES, CW), jnp.float32),
            pltpu.SemaphoreType.DMA,
        ],
    )
    def _sc1(xall, src4, dstb, out, agg_sh, zb, si, di, msg, gsem):
        cid = lax.axis_index("c")
        sid = lax.axis_index("s")
        _zero_fill(zb, CW)
        row0 = sid * R_MAIN
        rex = NS * R_MAIN + sid * 8
        blk0 = sid * B_MAIN1
        for cc in range(2):
            chunk = cid * 2 + cc
            for r in range(R_MAIN // ZROWS):
                pltpu.sync_copy(zb, agg_sh.at[pl.ds(row0 + r * ZROWS, ZROWS)])

            @pl.when(sid < R_EX_TILES)
            def _():
                pltpu.sync_copy(zb.at[pl.ds(0, 8)], agg_sh.at[pl.ds(rex, 8)])

            plsc.subcore_barrier()
            _edge_pass(
                xall, lambda off, n: src4.at[chunk, pl.ds(off, n)], dstb,
                agg_sh, si, di, msg, gsem,
                blk0, NG1, sid < EX8_1, EX8_OFF1 + sid * 8,
                sid == NS - 1, EX2_OFF)
            plsc.subcore_barrier()
            pltpu.sync_copy(agg_sh.at[pl.ds(row0, R_MAIN)],
                            out.at[chunk, pl.ds(row0, R_MAIN)])

            @pl.when(sid < R_EX_TILES)
            def _():
                pltpu.sync_copy(agg_sh.at[pl.ds(rex, 8)],
                                out.at[chunk, pl.ds(rex, 8)])

            plsc.subcore_barrier()

    @functools.partial(
        pl.kernel,
        out_type=jax.ShapeDtypeStruct((NC, N, K2), jnp.float32),
        mesh=mesh,
        compiler_params=params,
        scratch_types=[
            pltpu.VMEM_SHARED((N, K2), jnp.float32),
            pltpu.VMEM((ZROWS, K2), jnp.float32),
            pltpu.VMEM((G1, LANES), jnp.int32),
            pltpu.VMEM((G1, LANES), jnp.int32),
            pltpu.VMEM((SB, LANES, K2), jnp.float32),
            pltpu.SemaphoreType.DMA,
        ],
    )
    def _sc2(t16, srcb, dstb, out, agg_sh, zb, si, di, msg, gsem):
        cid = lax.axis_index("c")
        sid = lax.axis_index("s")
        _zero_fill(zb, K2)
        row0 = sid * R_MAIN
        rex = NS * R_MAIN + sid * 8
        blk0 = cid * HB2 + sid * B_MAIN2
        for r in range(R_MAIN // ZROWS):
            pltpu.sync_copy(zb, agg_sh.at[pl.ds(row0 + r * ZROWS, ZROWS)])

        @pl.when(sid < R_EX_TILES)
        def _():
            pltpu.sync_copy(zb.at[pl.ds(0, 8)], agg_sh.at[pl.ds(rex, 8)])

        plsc.subcore_barrier()
        _edge_pass(
            t16, lambda off, n: srcb.at[pl.ds(off, n)], dstb,
            agg_sh, si, di, msg, gsem,
            blk0, NG2, sid < 7 - cid, cid * HB2 + EX8_OFF2 + sid * 8,
            jnp.logical_and(cid == 1, sid == NS - 1), EX2_OFF)
        plsc.subcore_barrier()
        pltpu.sync_copy(agg_sh.at[pl.ds(row0, R_MAIN)],
                        out.at[cid, pl.ds(row0, R_MAIN)])

        @pl.when(sid < R_EX_TILES)
        def _():
            pltpu.sync_copy(agg_sh.at[pl.ds(rex, 8)],
                            out.at[cid, pl.ds(rex, 8)])

    return _sc1, _sc2


def _tca_body(agg_ref, x_ref, lp_ref, w1l_ref, w1r_ref, b1_ref,
              h_ref, xm_ref, ps_ref, psq_ref, m_ref):
    m128 = jax.nn.sigmoid(lp_ref[...])
    xm = x_ref[...] * m128[:, :D]
    xm_ref[...] = xm
    cnt = jnp.maximum(agg_ref[3, :, CNT_COL:CNT_COL + 1], 1.0)
    aggc = jnp.concatenate(
        [agg_ref[0], agg_ref[1], agg_ref[2], agg_ref[3]], axis=1)
    mean_m = aggc * m128 / cnt
    h = (jnp.dot(mean_m, w1l_ref[...], preferred_element_type=jnp.float32)
         + b1_ref[...]
         + jnp.dot(xm, w1r_ref[...], preferred_element_type=jnp.float32))
    h_ref[...] = h
    ps_ref[...] = jnp.sum(h, axis=0, keepdims=True)[:, None, :]
    psq_ref[...] = jnp.sum(h * h, axis=0, keepdims=True)[:, None, :]
    m_ref[...] = m128[:, :D]


def _tcb_body(h_ref, ps_ref, psq_ref, g_ref, bb_ref, w2l_ref, w2r_ref, b2_ref,
              t_ref, u_ref):
    s = jnp.sum(ps_ref[...], axis=0)
    sq = jnp.sum(psq_ref[...], axis=0)
    mu = s * (1.0 / N)
    var = sq * (1.0 / N) - mu * mu
    inv = lax.rsqrt(var + 1e-5)
    hn = (h_ref[...] - mu) * (inv * g_ref[...]) + bb_ref[...]
    he = jnp.where(hn > 0, hn, jnp.exp(hn) - 1.0)
    t_ref[...] = jnp.dot(he, w2l_ref[...], preferred_element_type=jnp.float32)
    u_ref[...] = (jnp.dot(he, w2r_ref[...], preferred_element_type=jnp.float32)
                  + b2_ref[...])


def _tcc_body(p2_ref, u_ref, agg_ref, o_ref):
    cnt = jnp.maximum(agg_ref[0, :, CNT_COL:CNT_COL + 1], 1.0)
    ssum = p2_ref[0] + p2_ref[1]
    o_ref[...] = ssum[:, :O] / cnt + u_ref[:, :O]


def kernel(x, edge_index, logits, W1l, b1, W1r, bn_gamma, bn_beta, W2l, b2, W2r):
    f32 = jnp.float32
    src = edge_index[0]
    dst = edge_index[1]

    xp = jnp.concatenate(
        [x, jnp.ones((N, 1), f32), jnp.zeros((N, DP - D - 1), f32)], axis=1)
    xall = xp.reshape(N, CH, CW).transpose(1, 0, 2).reshape(CH * N, CW)
    src4 = (src[None, :]
            + (jnp.arange(CH, dtype=jnp.int32) * N)[:, None]).reshape(CH, EB, LANES)
    srcb = src.reshape(EB, LANES)
    dstb = dst.reshape(EB, LANES)

    _sc1, _sc2 = _build_sc_kernels()
    agg4 = _sc1(xall, src4, dstb)

    w1l_pad = jnp.concatenate([W1l, jnp.zeros((DP - D, H), f32)], axis=0)
    lp = jnp.concatenate([logits, jnp.full((DP - D,), -1e30, f32)]).reshape(1, DP)

    h_pre, xm, ps, psq, m_out = pl.pallas_call(
        _tca_body,
        grid=(GQ,),
        in_specs=[
            pl.BlockSpec((CH, BN, CW), lambda i: (0, i, 0)),
            pl.BlockSpec((BN, D), lambda i: (i, 0)),
            pl.BlockSpec((1, DP), lambda i: (0, 0)),
            pl.BlockSpec((DP, H), lambda i: (0, 0)),
            pl.BlockSpec((D, H), lambda i: (0, 0)),
            pl.BlockSpec((1, H), lambda i: (0, 0)),
        ],
        out_specs=[
            pl.BlockSpec((BN, H), lambda i: (i, 0)),
            pl.BlockSpec((BN, D), lambda i: (i, 0)),
            pl.BlockSpec((1, 1, H), lambda i: (i, 0, 0)),
            pl.BlockSpec((1, 1, H), lambda i: (i, 0, 0)),
            pl.BlockSpec((1, D), lambda i: (0, 0)),
        ],
        out_shape=[
            jax.ShapeDtypeStruct((N, H), f32),
            jax.ShapeDtypeStruct((N, D), f32),
            jax.ShapeDtypeStruct((GQ, 1, H), f32),
            jax.ShapeDtypeStruct((GQ, 1, H), f32),
            jax.ShapeDtypeStruct((1, D), f32),
        ],
    )(agg4, x, lp, w1l_pad, W1r, b1.reshape(1, H))

    w2l_pad = jnp.concatenate([W2l, jnp.zeros((H, K2 - O), f32)], axis=1)
    w2r_pad = jnp.concatenate([W2r, jnp.zeros((H, K2 - O), f32)], axis=1)
    b2_pad = jnp.concatenate([b2, jnp.zeros((K2 - O,), f32)]).reshape(1, K2)

    t16, u16 = pl.pallas_call(
        _tcb_body,
        grid=(GQ,),
        in_specs=[
            pl.BlockSpec((BN, H), lambda i: (i, 0)),
            pl.BlockSpec((GQ, 1, H), lambda i: (0, 0, 0)),
            pl.BlockSpec((GQ, 1, H), lambda i: (0, 0, 0)),
            pl.BlockSpec((1, H), lambda i: (0, 0)),
            pl.BlockSpec((1, H), lambda i: (0, 0)),
            pl.BlockSpec((H, K2), lambda i: (0, 0)),
            pl.BlockSpec((H, K2), lambda i: (0, 0)),
            pl.BlockSpec((1, K2), lambda i: (0, 0)),
        ],
        out_specs=[
            pl.BlockSpec((BN, K2), lambda i: (i, 0)),
            pl.BlockSpec((BN, K2), lambda i: (i, 0)),
        ],
        out_shape=[
            jax.ShapeDtypeStruct((N, K2), f32),
            jax.ShapeDtypeStruct((N, K2), f32),
        ],
    )(h_pre, ps, psq, bn_gamma.reshape(1, H), bn_beta.reshape(1, H),
      w2l_pad, w2r_pad, b2_pad)

    p2 = _sc2(t16, srcb, dstb)

    pred = pl.pallas_call(
        _tcc_body,
        grid=(GQ,),
        in_specs=[
            pl.BlockSpec((NC, BN, K2), lambda i: (0, i, 0)),
            pl.BlockSpec((BN, K2), lambda i: (i, 0)),
            pl.BlockSpec((1, BN, CW), lambda i: (3, i, 0)),
        ],
        out_specs=pl.BlockSpec((BN, O), lambda i: (i, 0)),
        out_shape=jax.ShapeDtypeStruct((N, O), f32),
    )(p2, u16, agg4)

    return (pred, xm, m_out.reshape(D))


# speedup vs baseline: 10.0316x; 1.1417x over previous
import functools

import jax
import jax.numpy as jnp
from jax import lax
from jax.experimental import pallas as pl
from jax.experimental.pallas import tpu as pltpu
from jax.experimental.pallas import tpu_sc as plsc

N = 50000
E = 800000
D = 100
H = 128
O = 2

NC = 2
NS = 16
LANES = 128
EB = E // LANES
CH = 4
CW = 32
DP = CH * CW
CNT_COL = 4
K2 = 16
R_MAIN = 3120
R_EX_TILES = (N - NS * R_MAIN) // 8
ZROWS = 208

WAVE = 2
SB = 2 * WAVE
G1 = 24
NW = G1 // WAVE

B_MAIN1 = 384
NG1 = B_MAIN1 // G1
EX8_1 = 13
EX8_OFF1 = NS * B_MAIN1
EX2_OFF = 6248

HB2 = 3128
B_MAIN2 = 192
NG2 = B_MAIN2 // G1
EX8_OFF2 = NS * B_MAIN2

BN = 1000
GQ = N // BN

def _zero_fill(zb, width):
    z16 = jnp.zeros((16,), jnp.float32)

    def body(i, _):
        for w in range(width // 16):
            zb[i, w * 16:(w + 1) * 16] = z16
        return 0

    lax.fori_loop(0, ZROWS, body, 0)


def _run_blocks(table, idx_src, dst_ref, agg_sh, si, di, msg, gsem, off, nblk):
    pltpu.sync_copy(idx_src(off, nblk), si.at[pl.ds(0, nblk)])
    pltpu.sync_copy(dst_ref.at[pl.ds(off, nblk)], di.at[pl.ds(0, nblk)])
    for w0 in range(0, nblk, SB):
        wn = min(SB, nblk - w0)
        descs = [
            pltpu.async_copy(table.at[si.at[w0 + j]], msg.at[j], gsem)
            for j in range(wn)
        ]
        for d in descs:
            d.wait()
        for j in range(wn):
            pltpu.sync_copy(msg.at[j], agg_sh.at[di.at[w0 + j]], add=True)


def _edge_pass(table, idx_src, dst_ref, agg_sh, si, di, msg, gsem,
               blk0, ngroups, ex8_pred, ex8_off, ex2_pred, ex2_off):
    def fire(half, w):
        for j in range(WAVE):
            pltpu.async_copy(table.at[si.at[WAVE * w + j]],
                             msg.at[half * WAVE + j], gsem)

    def drain_scatter(half, w):
        for j in range(WAVE):
            pltpu.make_async_copy(table.at[si.at[WAVE * w + j]],
                                  msg.at[half * WAVE + j], gsem).wait()
        for j in range(WAVE):
            pltpu.sync_copy(msg.at[half * WAVE + j],
                            agg_sh.at[di.at[WAVE * w + j]], add=True)

    def group(k, _):
        gb = blk0 + k * G1
        pltpu.sync_copy(idx_src(gb, G1), si)
        pltpu.sync_copy(dst_ref.at[pl.ds(gb, G1)], di)
        fire(0, 0)

        def waves(t, _):
            w0 = 2 * t
            fire(1, w0 + 1)
            drain_scatter(0, w0)

            @pl.when(t < NW // 2 - 1)
            def _():
                fire(0, w0 + 2)

            drain_scatter(1, w0 + 1)
            return 0

        lax.fori_loop(0, NW // 2, waves, 0)
        return 0

    lax.fori_loop(0, ngroups, group, 0)

    @pl.when(ex8_pred)
    def _():
        _run_blocks(table, idx_src, dst_ref, agg_sh, si, di, msg, gsem,
                    ex8_off, 8)

    @pl.when(ex2_pred)
    def _():
        _run_blocks(table, idx_src, dst_ref, agg_sh, si, di, msg, gsem,
                    ex2_off, 2)


@functools.lru_cache(maxsize=1)
def _build_sc_kernels():
    mesh = plsc.VectorSubcoreMesh(
        core_axis_name="c", subcore_axis_name="s", num_cores=NC, num_subcores=NS)
    params = pltpu.CompilerParams(use_tc_tiling_on_sc=False)

    @functools.partial(
        pl.kernel,
        out_type=jax.ShapeDtypeStruct((CH, N, CW), jnp.float32),
        mesh=mesh,
        compiler_params=params,
        scratch_types=[
            pltpu.VMEM_SHARED((N, CW), jnp.float32),
            pltpu.VMEM((ZROWS, CW), jnp.float32),
            pltpu.VMEM((G1, LANES), jnp.int32),
            pltpu.VMEM((G1, LANES), jnp.int32),
            pltpu.VMEM((SB, LANES, CW), jnp.float32),
            pltpu.SemaphoreType.DMA,
        ],
    )
    def _sc1(xall, src4, dstb, out, agg_sh, zb, si, di, msg, gsem):
        cid = lax.axis_index("c")
        sid = lax.axis_index("s")
        _zero_fill(zb, CW)
        row0 = sid * R_MAIN
        rex = NS * R_MAIN + sid * 8
        blk0 = sid * B_MAIN1
        for cc in range(2):
            chunk = cid * 2 + cc
            for r in range(R_MAIN // ZROWS):
                pltpu.sync_copy(zb, agg_sh.at[pl.ds(row0 + r * ZROWS, ZROWS)])

            @pl.when(sid < R_EX_TILES)
            def _():
                pltpu.sync_copy(zb.at[pl.ds(0, 8)], agg_sh.at[pl.ds(rex, 8)])

            plsc.subcore_barrier()
            _edge_pass(
                xall, lambda off, n: src4.at[chunk, pl.ds(off, n)], dstb,
                agg_sh, si, di, msg, gsem,
                blk0, NG1, sid < EX8_1, EX8_OFF1 + sid * 8,
                sid == NS - 1, EX2_OFF)
            plsc.subcore_barrier()
            pltpu.sync_copy(agg_sh.at[pl.ds(row0, R_MAIN)],
                            out.at[chunk, pl.ds(row0, R_MAIN)])

            @pl.when(sid < R_EX_TILES)
            def _():
                pltpu.sync_copy(agg_sh.at[pl.ds(rex, 8)],
                                out.at[chunk, pl.ds(rex, 8)])

            plsc.subcore_barrier()

    @functools.partial(
        pl.kernel,
        out_type=jax.ShapeDtypeStruct((NC, N, K2), jnp.float32),
        mesh=mesh,
        compiler_params=params,
        scratch_types=[
            pltpu.VMEM_SHARED((N, K2), jnp.float32),
            pltpu.VMEM((ZROWS, K2), jnp.float32),
            pltpu.VMEM((G1, LANES), jnp.int32),
            pltpu.VMEM((G1, LANES), jnp.int32),
            pltpu.VMEM((SB, LANES, K2), jnp.float32),
            pltpu.SemaphoreType.DMA,
        ],
    )
    def _sc2(t16, srcb, dstb, out, agg_sh, zb, si, di, msg, gsem):
        cid = lax.axis_index("c")
        sid = lax.axis_index("s")
        _zero_fill(zb, K2)
        row0 = sid * R_MAIN
        rex = NS * R_MAIN + sid * 8
        blk0 = cid * HB2 + sid * B_MAIN2
        for r in range(R_MAIN // ZROWS):
            pltpu.sync_copy(zb, agg_sh.at[pl.ds(row0 + r * ZROWS, ZROWS)])

        @pl.when(sid < R_EX_TILES)
        def _():
            pltpu.sync_copy(zb.at[pl.ds(0, 8)], agg_sh.at[pl.ds(rex, 8)])

        plsc.subcore_barrier()
        _edge_pass(
            t16, lambda off, n: srcb.at[pl.ds(off, n)], dstb,
            agg_sh, si, di, msg, gsem,
            blk0, NG2, sid < 7 - cid, cid * HB2 + EX8_OFF2 + sid * 8,
            jnp.logical_and(cid == 1, sid == NS - 1), EX2_OFF)
        plsc.subcore_barrier()
        pltpu.sync_copy(agg_sh.at[pl.ds(row0, R_MAIN)],
                        out.at[cid, pl.ds(row0, R_MAIN)])

        @pl.when(sid < R_EX_TILES)
        def _():
            pltpu.sync_copy(agg_sh.at[pl.ds(rex, 8)],
                            out.at[cid, pl.ds(rex, 8)])

    return _sc1, _sc2


def _tca_body(agg_ref, x_ref, lp_ref, w1l_ref, w1r_ref, b1_ref,
              h_ref, xm_ref, ps_ref, psq_ref, m_ref):
    m128 = jax.nn.sigmoid(lp_ref[...])
    xm = x_ref[...] * m128[:, :D]
    xm_ref[...] = xm
    cnt = jnp.maximum(agg_ref[3, :, CNT_COL:CNT_COL + 1], 1.0)
    aggc = jnp.concatenate(
        [agg_ref[0], agg_ref[1], agg_ref[2], agg_ref[3]], axis=1)
    mean_m = aggc * m128 / cnt
    h = (jnp.dot(mean_m, w1l_ref[...], preferred_element_type=jnp.float32)
         + b1_ref[...]
         + jnp.dot(xm, w1r_ref[...], preferred_element_type=jnp.float32))
    h_ref[...] = h
    ps_ref[...] = jnp.sum(h, axis=0, keepdims=True)[:, None, :]
    psq_ref[...] = jnp.sum(h * h, axis=0, keepdims=True)[:, None, :]
    m_ref[...] = m128[:, :D]


def _tcb_body(h_ref, ps_ref, psq_ref, g_ref, bb_ref, w2l_ref, w2r_ref, b2_ref,
              t_ref, u_ref):
    s = jnp.sum(ps_ref[...], axis=0)
    sq = jnp.sum(psq_ref[...], axis=0)
    mu = s * (1.0 / N)
    var = sq * (1.0 / N) - mu * mu
    inv = lax.rsqrt(var + 1e-5)
    hn = (h_ref[...] - mu) * (inv * g_ref[...]) + bb_ref[...]
    he = jnp.where(hn > 0, hn, jnp.exp(hn) - 1.0)
    t_ref[...] = jnp.dot(he, w2l_ref[...], preferred_element_type=jnp.float32)
    u_ref[...] = (jnp.dot(he, w2r_ref[...], preferred_element_type=jnp.float32)
                  + b2_ref[...])


def _tcc_body(p2_ref, u_ref, agg_ref, o_ref):
    cnt = jnp.maximum(agg_ref[0, :, CNT_COL:CNT_COL + 1], 1.0)
    ssum = p2_ref[0] + p2_ref[1]
    o_ref[...] = ssum[:, :O] / cnt + u_ref[:, :O]


def kernel(x, edge_index, logits, W1l, b1, W1r, bn_gamma, bn_beta, W2l, b2, W2r):
    f32 = jnp.float32
    src = edge_index[0]
    dst = edge_index[1]

    xp = jnp.concatenate(
        [x, jnp.ones((N, 1), f32), jnp.zeros((N, DP - D - 1), f32)], axis=1)
    xall = xp.reshape(N, CH, CW).transpose(1, 0, 2).reshape(CH * N, CW)
    src4 = (src[None, :]
            + (jnp.arange(CH, dtype=jnp.int32) * N)[:, None]).reshape(CH, EB, LANES)
    srcb = src.reshape(EB, LANES)
    dstb = dst.reshape(EB, LANES)

    _sc1, _sc2 = _build_sc_kernels()
    agg4 = _sc1(xall, src4, dstb)

    w1l_pad = jnp.concatenate([W1l, jnp.zeros((DP - D, H), f32)], axis=0)
    lp = jnp.concatenate([logits, jnp.full((DP - D,), -1e30, f32)]).reshape(1, DP)

    h_pre, xm, ps, psq, m_out = pl.pallas_call(
        _tca_body,
        grid=(GQ,),
        in_specs=[
            pl.BlockSpec((CH, BN, CW), lambda i: (0, i, 0)),
            pl.BlockSpec((BN, D), lambda i: (i, 0)),
            pl.BlockSpec((1, DP), lambda i: (0, 0)),
            pl.BlockSpec((DP, H), lambda i: (0, 0)),
            pl.BlockSpec((D, H), lambda i: (0, 0)),
            pl.BlockSpec((1, H), lambda i: (0, 0)),
        ],
        out_specs=[
            pl.BlockSpec((BN, H), lambda i: (i, 0)),
            pl.BlockSpec((BN, D), lambda i: (i, 0)),
            pl.BlockSpec((1, 1, H), lambda i: (i, 0, 0)),
            pl.BlockSpec((1, 1, H), lambda i: (i, 0, 0)),
            pl.BlockSpec((1, D), lambda i: (0, 0)),
        ],
        out_shape=[
            jax.ShapeDtypeStruct((N, H), f32),
            jax.ShapeDtypeStruct((N, D), f32),
            jax.ShapeDtypeStruct((GQ, 1, H), f32),
            jax.ShapeDtypeStruct((GQ, 1, H), f32),
            jax.ShapeDtypeStruct((1, D), f32),
        ],
    )(agg4, x, lp, w1l_pad, W1r, b1.reshape(1, H))

    w2l_pad = jnp.concatenate([W2l, jnp.zeros((H, K2 - O), f32)], axis=1)
    w2r_pad = jnp.concatenate([W2r, jnp.zeros((H, K2 - O), f32)], axis=1)
    b2_pad = jnp.concatenate([b2, jnp.zeros((K2 - O,), f32)]).reshape(1, K2)

    t16, u16 = pl.pallas_call(
        _tcb_body,
        grid=(GQ,),
        in_specs=[
            pl.BlockSpec((BN, H), lambda i: (i, 0)),
            pl.BlockSpec((GQ, 1, H), lambda i: (0, 0, 0)),
            pl.BlockSpec((GQ, 1, H), lambda i: (0, 0, 0)),
            pl.BlockSpec((1, H), lambda i: (0, 0)),
            pl.BlockSpec((1, H), lambda i: (0, 0)),
            pl.BlockSpec((H, K2), lambda i: (0, 0)),
            pl.BlockSpec((H, K2), lambda i: (0, 0)),
            pl.BlockSpec((1, K2), lambda i: (0, 0)),
        ],
        out_specs=[
            pl.BlockSpec((BN, K2), lambda i: (i, 0)),
            pl.BlockSpec((BN, K2), lambda i: (i, 0)),
        ],
        out_shape=[
            jax.ShapeDtypeStruct((N, K2), f32),
            jax.ShapeDtypeStruct((N, K2), f32),
        ],
    )(h_pre, ps, psq, bn_gamma.reshape(1, H), bn_beta.reshape(1, H),
      w2l_pad, w2r_pad, b2_pad)

    p2 = _sc2(t16, srcb, dstb)

    pred = pl.pallas_call(
        _tcc_body,
        grid=(GQ,),
        in_specs=[
            pl.BlockSpec((NC, BN, K2), lambda i: (0, i, 0)),
            pl.BlockSpec((BN, K2), lambda i: (i, 0)),
            pl.BlockSpec((1, BN, CW), lambda i: (3, i, 0)),
        ],
        out_specs=pl.BlockSpec((BN, O), lambda i: (i, 0)),
        out_shape=jax.ShapeDtypeStruct((N, O), f32),
    )(p2, u16, agg4)

    return (pred, xm, m_out.reshape(D))


# speedup vs baseline: 11.3736x; 1.1338x over previous
import functools

import jax
import jax.numpy as jnp
from jax import lax
from jax.experimental import pallas as pl
from jax.experimental.pallas import tpu as pltpu
from jax.experimental.pallas import tpu_sc as plsc

N = 50000
E = 800000
D = 100
H = 128
O = 2

NC = 2
NS = 16
LANES = 128
EB = E // LANES
CH = 4
CW = 32
DP = CH * CW
K2 = 16
CNT2 = 2
R_MAIN = 3120
R_EX_TILES = (N - NS * R_MAIN) // 8
ZROWS = 208

WAVE = 2
SB = 2 * WAVE
G1 = 24
NW = G1 // WAVE

B_MAIN1 = 384
NG1 = B_MAIN1 // G1
EX8_1 = 13
EX8_OFF1 = NS * B_MAIN1
EX2_OFF = 6248

HB2 = 3128
B_MAIN2 = 192
NG2 = B_MAIN2 // G1
EX8_OFF2 = NS * B_MAIN2

BN = 1000
GQ = N // BN

def _zero_fill(zb, width):
    z16 = jnp.zeros((16,), jnp.float32)

    def body(i, _):
        for w in range(width // 16):
            zb[i, w * 16:(w + 1) * 16] = z16
        return 0

    lax.fori_loop(0, ZROWS, body, 0)


def _adjust_idx(si, nrows, scale, offset):
    if offset is None:
        return

    def row(r, _):
        for v in range(LANES // 16):
            sl = pl.ds(v * 16, 16)
            si[r, sl] = si[r, sl] * scale + offset
        return 0

    lax.fori_loop(0, nrows, row, 0)


def _run_blocks(table, idx_src, dst_ref, agg_sh, si, di, msg, gsem, off, nblk,
                scale, offset):
    pltpu.sync_copy(idx_src(off, nblk), si.at[pl.ds(0, nblk)])
    pltpu.sync_copy(dst_ref.at[pl.ds(off, nblk)], di.at[pl.ds(0, nblk)])
    _adjust_idx(si, nblk, scale, offset)
    for w0 in range(0, nblk, SB):
        wn = min(SB, nblk - w0)
        descs = [
            pltpu.async_copy(table.at[si.at[w0 + j]], msg.at[j], gsem)
            for j in range(wn)
        ]
        for d in descs:
            d.wait()
        for j in range(wn):
            pltpu.sync_copy(msg.at[j], agg_sh.at[di.at[w0 + j]], add=True)


def _edge_pass(table, idx_src, dst_ref, agg_sh, si, di, msg, gsem,
               blk0, ngroups, ex8_pred, ex8_off, ex2_pred, ex2_off,
               scale=1, offset=None):
    def fire(half, w):
        for j in range(WAVE):
            pltpu.async_copy(table.at[si.at[WAVE * w + j]],
                             msg.at[half * WAVE + j], gsem)

    def drain_scatter(half, w):
        for j in range(WAVE):
            pltpu.make_async_copy(table.at[si.at[WAVE * w + j]],
                                  msg.at[half * WAVE + j], gsem).wait()
        for j in range(WAVE):
            pltpu.sync_copy(msg.at[half * WAVE + j],
                            agg_sh.at[di.at[WAVE * w + j]], add=True)

    def group(k, _):
        gb = blk0 + k * G1
        pltpu.sync_copy(idx_src(gb, G1), si)
        pltpu.sync_copy(dst_ref.at[pl.ds(gb, G1)], di)
        _adjust_idx(si, G1, scale, offset)
        fire(0, 0)

        def waves(t, _):
            w0 = 2 * t
            fire(1, w0 + 1)
            drain_scatter(0, w0)

            @pl.when(t < NW // 2 - 1)
            def _():
                fire(0, w0 + 2)

            drain_scatter(1, w0 + 1)
            return 0

        lax.fori_loop(0, NW // 2, waves, 0)
        return 0

    lax.fori_loop(0, ngroups, group, 0)

    @pl.when(ex8_pred)
    def _():
        _run_blocks(table, idx_src, dst_ref, agg_sh, si, di, msg, gsem,
                    ex8_off, 8, scale, offset)

    @pl.when(ex2_pred)
    def _():
        _run_blocks(table, idx_src, dst_ref, agg_sh, si, di, msg, gsem,
                    ex2_off, 2, scale, offset)


@functools.lru_cache(maxsize=1)
def _build_sc_kernels():
    mesh = plsc.VectorSubcoreMesh(
        core_axis_name="c", subcore_axis_name="s", num_cores=NC, num_subcores=NS)
    params = pltpu.CompilerParams(use_tc_tiling_on_sc=False)

    @functools.partial(
        pl.kernel,
        out_type=jax.ShapeDtypeStruct((N, DP), jnp.float32),
        mesh=mesh,
        compiler_params=params,
        scratch_types=[
            pltpu.VMEM_SHARED((N, CW), jnp.float32),
            pltpu.VMEM((ZROWS, CW), jnp.float32),
            pltpu.VMEM((G1, LANES), jnp.int32),
            pltpu.VMEM((G1, LANES), jnp.int32),
            pltpu.VMEM((SB, LANES, CW), jnp.float32),
            pltpu.SemaphoreType.DMA,
        ],
    )
    def _sc1(xall, srcb, dstb, out, agg_sh, zb, si, di, msg, gsem):
        cid = lax.axis_index("c")
        sid = lax.axis_index("s")
        _zero_fill(zb, CW)
        row0 = sid * R_MAIN
        rex = NS * R_MAIN + sid * 8
        blk0 = sid * B_MAIN1
        for cc in range(2):
            chunk = cid * 2 + cc
            for r in range(R_MAIN // ZROWS):
                pltpu.sync_copy(zb, agg_sh.at[pl.ds(row0 + r * ZROWS, ZROWS)])

            @pl.when(sid < R_EX_TILES)
            def _():
                pltpu.sync_copy(zb.at[pl.ds(0, 8)], agg_sh.at[pl.ds(rex, 8)])

            plsc.subcore_barrier()
            _edge_pass(
                xall, lambda off, n: srcb.at[pl.ds(off, n)], dstb,
                agg_sh, si, di, msg, gsem,
                blk0, NG1, sid < EX8_1, EX8_OFF1 + sid * 8,
                sid == NS - 1, EX2_OFF,
                scale=CH, offset=chunk)
            plsc.subcore_barrier()
            pltpu.sync_copy(agg_sh.at[pl.ds(row0, R_MAIN)],
                            out.at[pl.ds(row0, R_MAIN), pl.ds(chunk * CW, CW)])

            @pl.when(sid < R_EX_TILES)
            def _():
                pltpu.sync_copy(agg_sh.at[pl.ds(rex, 8)],
                                out.at[pl.ds(rex, 8), pl.ds(chunk * CW, CW)])

            plsc.subcore_barrier()

    @functools.partial(
        pl.kernel,
        out_type=jax.ShapeDtypeStruct((N, NC * K2), jnp.float32),
        mesh=mesh,
        compiler_params=params,
        scratch_types=[
            pltpu.VMEM_SHARED((N, K2), jnp.float32),
            pltpu.VMEM((ZROWS, K2), jnp.float32),
            pltpu.VMEM((G1, LANES), jnp.int32),
            pltpu.VMEM((G1, LANES), jnp.int32),
            pltpu.VMEM((SB, LANES, K2), jnp.float32),
            pltpu.SemaphoreType.DMA,
        ],
    )
    def _sc2(t16, srcb, dstb, out, agg_sh, zb, si, di, msg, gsem):
        cid = lax.axis_index("c")
        sid = lax.axis_index("s")
        _zero_fill(zb, K2)
        row0 = sid * R_MAIN
        rex = NS * R_MAIN + sid * 8
        blk0 = cid * HB2 + sid * B_MAIN2
        for r in range(R_MAIN // ZROWS):
            pltpu.sync_copy(zb, agg_sh.at[pl.ds(row0 + r * ZROWS, ZROWS)])

        @pl.when(sid < R_EX_TILES)
        def _():
            pltpu.sync_copy(zb.at[pl.ds(0, 8)], agg_sh.at[pl.ds(rex, 8)])

        plsc.subcore_barrier()
        _edge_pass(
            t16, lambda off, n: srcb.at[pl.ds(off, n)], dstb,
            agg_sh, si, di, msg, gsem,
            blk0, NG2, sid < 7 - cid, cid * HB2 + EX8_OFF2 + sid * 8,
            jnp.logical_and(cid == 1, sid == NS - 1), EX2_OFF)
        plsc.subcore_barrier()
        pltpu.sync_copy(agg_sh.at[pl.ds(row0, R_MAIN)],
                        out.at[pl.ds(row0, R_MAIN), pl.ds(cid * K2, K2)])

        @pl.when(sid < R_EX_TILES)
        def _():
            pltpu.sync_copy(agg_sh.at[pl.ds(rex, 8)],
                            out.at[pl.ds(rex, 8), pl.ds(cid * K2, K2)])

    return _sc1, _sc2


def _tcp_body(x_ref, lp_ref, xp_ref, xm_ref, m_ref):
    x = x_ref[...]
    m128 = jax.nn.sigmoid(lp_ref[...])
    xm_ref[...] = x * m128[:, :D]
    xp_ref[...] = jnp.concatenate(
        [x, jnp.ones((BN, 1), jnp.float32), jnp.zeros((BN, DP - D - 1),
                                                      jnp.float32)], axis=1)
    m_ref[...] = m128[:, :D]


def _tca_body(agg_ref, xp_ref, lp_ref, w1l_ref, w1r_ref, b1_ref,
              h_ref, ps_ref, psq_ref):
    m128 = jax.nn.sigmoid(lp_ref[...])
    aggc = agg_ref[...]
    cnt = jnp.maximum(aggc[:, D:D + 1], 1.0)
    mean_m = aggc * m128 / cnt
    xmm = xp_ref[...] * m128
    h = (jnp.dot(mean_m, w1l_ref[...], preferred_element_type=jnp.float32)
         + b1_ref[...]
         + jnp.dot(xmm, w1r_ref[...], preferred_element_type=jnp.float32))
    h_ref[...] = h
    ps_ref[...] = jnp.sum(h, axis=0, keepdims=True)[:, None, :]
    psq_ref[...] = jnp.sum(h * h, axis=0, keepdims=True)[:, None, :]


def _tcb_body(h_ref, ps_ref, psq_ref, g_ref, bb_ref, w2l_ref, w2r_ref, b2_ref,
              t_ref, u_ref):
    s = jnp.sum(ps_ref[...], axis=0)
    sq = jnp.sum(psq_ref[...], axis=0)
    mu = s * (1.0 / N)
    var = sq * (1.0 / N) - mu * mu
    inv = lax.rsqrt(var + 1e-5)
    hn = (h_ref[...] - mu) * (inv * g_ref[...]) + bb_ref[...]
    he = jnp.where(hn > 0, hn, jnp.exp(hn) - 1.0)
    ones_col = jnp.float32(1.0) * (jax.lax.broadcasted_iota(
        jnp.int32, (BN, K2), 1) == CNT2).astype(jnp.float32)
    t_ref[...] = (jnp.dot(he, w2l_ref[...], preferred_element_type=jnp.float32)
                  + ones_col)
    u_ref[...] = (jnp.dot(he, w2r_ref[...], preferred_element_type=jnp.float32)
                  + b2_ref[...])


def _tcc_body(p2_ref, u_ref, o_ref):
    p2 = p2_ref[...]
    ssum = p2[:, :K2] + p2[:, K2:]
    cnt = jnp.maximum(ssum[:, CNT2:CNT2 + 1], 1.0)
    o_ref[...] = ssum[:, :O] / cnt + u_ref[:, :O]


def kernel(x, edge_index, logits, W1l, b1, W1r, bn_gamma, bn_beta, W2l, b2, W2r):
    f32 = jnp.float32
    src = edge_index[0]
    dst = edge_index[1]

    lp = jnp.concatenate([logits, jnp.full((DP - D,), -1e30, f32)]).reshape(1, DP)

    xp, xm, m_out = pl.pallas_call(
        _tcp_body,
        grid=(GQ,),
        in_specs=[
            pl.BlockSpec((BN, D), lambda i: (i, 0)),
            pl.BlockSpec((1, DP), lambda i: (0, 0)),
        ],
        out_specs=[
            pl.BlockSpec((BN, DP), lambda i: (i, 0)),
            pl.BlockSpec((BN, D), lambda i: (i, 0)),
            pl.BlockSpec((1, D), lambda i: (0, 0)),
        ],
        out_shape=[
            jax.ShapeDtypeStruct((N, DP), f32),
            jax.ShapeDtypeStruct((N, D), f32),
            jax.ShapeDtypeStruct((1, D), f32),
        ],
    )(x, lp)

    xall = xp.reshape(CH * N, CW)
    srcb = src.reshape(EB, LANES)
    dstb = dst.reshape(EB, LANES)

    _sc1, _sc2 = _build_sc_kernels()
    agg = _sc1(xall, srcb, dstb)

    w1l_pad = jnp.concatenate([W1l, jnp.zeros((DP - D, H), f32)], axis=0)
    w1r_pad = jnp.concatenate([W1r, jnp.zeros((DP - D, H), f32)], axis=0)

    h_pre, ps, psq = pl.pallas_call(
        _tca_body,
        grid=(GQ,),
        in_specs=[
            pl.BlockSpec((BN, DP), lambda i: (i, 0)),
            pl.BlockSpec((BN, DP), lambda i: (i, 0)),
            pl.BlockSpec((1, DP), lambda i: (0, 0)),
            pl.BlockSpec((DP, H), lambda i: (0, 0)),
            pl.BlockSpec((DP, H), lambda i: (0, 0)),
            pl.BlockSpec((1, H), lambda i: (0, 0)),
        ],
        out_specs=[
            pl.BlockSpec((BN, H), lambda i: (i, 0)),
            pl.BlockSpec((1, 1, H), lambda i: (i, 0, 0)),
            pl.BlockSpec((1, 1, H), lambda i: (i, 0, 0)),
        ],
        out_shape=[
            jax.ShapeDtypeStruct((N, H), f32),
            jax.ShapeDtypeStruct((GQ, 1, H), f32),
            jax.ShapeDtypeStruct((GQ, 1, H), f32),
        ],
    )(agg, xp, lp, w1l_pad, w1r_pad, b1.reshape(1, H))

    w2l_pad = jnp.concatenate([W2l, jnp.zeros((H, K2 - O), f32)], axis=1)
    w2r_pad = jnp.concatenate([W2r, jnp.zeros((H, K2 - O), f32)], axis=1)
    b2_pad = jnp.concatenate([b2, jnp.zeros((K2 - O,), f32)]).reshape(1, K2)

    t16, u16 = pl.pallas_call(
        _tcb_body,
        grid=(GQ,),
        in_specs=[
            pl.BlockSpec((BN, H), lambda i: (i, 0)),
            pl.BlockSpec((GQ, 1, H), lambda i: (0, 0, 0)),
            pl.BlockSpec((GQ, 1, H), lambda i: (0, 0, 0)),
            pl.BlockSpec((1, H), lambda i: (0, 0)),
            pl.BlockSpec((1, H), lambda i: (0, 0)),
            pl.BlockSpec((H, K2), lambda i: (0, 0)),
            pl.BlockSpec((H, K2), lambda i: (0, 0)),
            pl.BlockSpec((1, K2), lambda i: (0, 0)),
        ],
        out_specs=[
            pl.BlockSpec((BN, K2), lambda i: (i, 0)),
            pl.BlockSpec((BN, K2), lambda i: (i, 0)),
        ],
        out_shape=[
            jax.ShapeDtypeStruct((N, K2), f32),
            jax.ShapeDtypeStruct((N, K2), f32),
        ],
    )(h_pre, ps, psq, bn_gamma.reshape(1, H), bn_beta.reshape(1, H),
      w2l_pad, w2r_pad, b2_pad)

    p2 = _sc2(t16, srcb, dstb)

    pred = pl.pallas_call(
        _tcc_body,
        grid=(GQ,),
        in_specs=[
            pl.BlockSpec((BN, NC * K2), lambda i: (i, 0)),
            pl.BlockSpec((BN, K2), lambda i: (i, 0)),
        ],
        out_specs=pl.BlockSpec((BN, O), lambda i: (i, 0)),
        out_shape=jax.ShapeDtypeStruct((N, O), f32),
    )(p2, u16)

    return (pred, xm, m_out.reshape(D))


# speedup vs baseline: 11.7641x; 1.0343x over previous
import functools

import jax
import jax.numpy as jnp
from jax import lax
from jax.experimental import pallas as pl
from jax.experimental.pallas import tpu as pltpu
from jax.experimental.pallas import tpu_sc as plsc

N = 50000
E = 800000
D = 100
H = 128
O = 2

NC = 2
NS = 16
LANES = 128
EB = E // LANES
CH = 4
CW = 32
DP = CH * CW
K2 = 16
CNT2 = 2
R_MAIN = 3120
R_EX_TILES = (N - NS * R_MAIN) // 8
ZROWS = 48

WAVE = 2
NBUF = 3
SB = NBUF * WAVE
G1 = 16
NW = G1 // WAVE

B_MAIN1 = 384
NG1 = B_MAIN1 // G1
EX8_1 = 13
EX8_OFF1 = NS * B_MAIN1
EX2_OFF = 6248

HB2 = 3128
B_MAIN2 = 192
NG2 = B_MAIN2 // G1
EX8_OFF2 = NS * B_MAIN2

BN = 1000
GQ = N // BN

def _zero_fill(zb, width):
    z16 = jnp.zeros((16,), jnp.float32)

    def body(i, _):
        for w in range(width // 16):
            zb[i, w * 16:(w + 1) * 16] = z16
        return 0

    lax.fori_loop(0, ZROWS, body, 0)


def _adjust_idx(si, nrows, scale, offset):
    if offset is None:
        return

    def row(r, _):
        for v in range(LANES // 16):
            sl = pl.ds(v * 16, 16)
            si[r, sl] = si[r, sl] * scale + offset
        return 0

    lax.fori_loop(0, nrows, row, 0)


def _zero_rows(agg_sh, zb, sem, row0, nrows):
    nz = nrows // ZROWS

    def fz(r, _):
        pltpu.async_copy(zb, agg_sh.at[pl.ds(row0 + r * ZROWS, ZROWS)], sem)
        return 0

    def wz(r, _):
        pltpu.make_async_copy(zb, agg_sh.at[pl.ds(row0, ZROWS)], sem).wait()
        return 0

    lax.fori_loop(0, nz, fz, 0)
    lax.fori_loop(0, nz, wz, 0)


def _run_blocks(table, idx_src, dst_ref, agg_sh, si, di, msg, gsem, off, nblk,
                scale, offset):
    pltpu.sync_copy(idx_src(off, nblk), si.at[pl.ds(0, nblk)])
    pltpu.sync_copy(dst_ref.at[pl.ds(off, nblk)], di.at[pl.ds(0, nblk)])
    _adjust_idx(si, nblk, scale, offset)
    for w0 in range(0, nblk, SB):
        wn = min(SB, nblk - w0)
        descs = [
            pltpu.async_copy(table.at[si.at[w0 + j]], msg.at[j], gsem)
            for j in range(wn)
        ]
        for d in descs:
            d.wait()
        for j in range(wn):
            pltpu.sync_copy(msg.at[j], agg_sh.at[di.at[w0 + j]], add=True)


def _edge_pass(table, idx_src, dst_ref, agg_sh, si, di, msg, gsem, ssem,
               blk0, ngroups, ex8_pred, ex8_off, ex2_pred, ex2_off,
               scale=1, offset=None):
    def fire_g(w):
        b = (w % NBUF) * WAVE
        for j in range(WAVE):
            pltpu.async_copy(table.at[si.at[WAVE * w + j]],
                             msg.at[b + j], gsem)

    def wait_g(w):
        b = (w % NBUF) * WAVE
        for j in range(WAVE):
            pltpu.make_async_copy(table.at[si.at[WAVE * w + j]],
                                  msg.at[b + j], gsem).wait()

    def fire_s(w):
        b = (w % NBUF) * WAVE
        for j in range(WAVE):
            pltpu.async_copy(msg.at[b + j], agg_sh.at[di.at[WAVE * w + j]],
                             ssem, add=True)

    def wait_s(w):
        b = (w % NBUF) * WAVE
        for j in range(WAVE):
            pltpu.make_async_copy(msg.at[b + j], agg_sh.at[di.at[WAVE * w + j]],
                                  ssem).wait()

    def group(k, _):
        gb = blk0 + k * G1
        pltpu.sync_copy(idx_src(gb, G1), si)
        pltpu.sync_copy(dst_ref.at[pl.ds(gb, G1)], di)
        _adjust_idx(si, G1, scale, offset)
        fire_g(0)
        fire_g(1)

        def waves(w, _):
            wait_g(w)
            fire_s(w)

            @pl.when(w >= 1)
            def _():
                wait_s(w - 1)

            @pl.when(w + 2 < NW)
            def _():
                fire_g(w + 2)

            return 0

        lax.fori_loop(0, NW, waves, 0)
        wait_s(NW - 1)
        return 0

    lax.fori_loop(0, ngroups, group, 0)

    @pl.when(ex8_pred)
    def _():
        _run_blocks(table, idx_src, dst_ref, agg_sh, si, di, msg, gsem,
                    ex8_off, 8, scale, offset)

    @pl.when(ex2_pred)
    def _():
        _run_blocks(table, idx_src, dst_ref, agg_sh, si, di, msg, gsem,
                    ex2_off, 2, scale, offset)


@functools.lru_cache(maxsize=1)
def _build_sc_kernels():
    mesh = plsc.VectorSubcoreMesh(
        core_axis_name="c", subcore_axis_name="s", num_cores=NC, num_subcores=NS)
    params = pltpu.CompilerParams(use_tc_tiling_on_sc=False)

    @functools.partial(
        pl.kernel,
        out_type=jax.ShapeDtypeStruct((N, DP), jnp.float32),
        mesh=mesh,
        compiler_params=params,
        scratch_types=[
            pltpu.VMEM_SHARED((N, CW), jnp.float32),
            pltpu.VMEM((ZROWS, CW), jnp.float32),
            pltpu.VMEM((G1, LANES), jnp.int32),
            pltpu.VMEM((G1, LANES), jnp.int32),
            pltpu.VMEM((SB, LANES, CW), jnp.float32),
            pltpu.SemaphoreType.DMA,
            pltpu.SemaphoreType.DMA,
        ],
    )
    def _sc1(xall, srcb, dstb, out, agg_sh, zb, si, di, msg, gsem, ssem):
        cid = lax.axis_index("c")
        sid = lax.axis_index("s")
        _zero_fill(zb, CW)
        row0 = sid * R_MAIN
        rex = NS * R_MAIN + sid * 8
        blk0 = sid * B_MAIN1
        for cc in range(2):
            chunk = cid * 2 + cc
            _zero_rows(agg_sh, zb, ssem, row0, R_MAIN)

            @pl.when(sid < R_EX_TILES)
            def _():
                pltpu.sync_copy(zb.at[pl.ds(0, 8)], agg_sh.at[pl.ds(rex, 8)])

            plsc.subcore_barrier()
            _edge_pass(
                xall, lambda off, n: srcb.at[pl.ds(off, n)], dstb,
                agg_sh, si, di, msg, gsem, ssem,
                blk0, NG1, sid < EX8_1, EX8_OFF1 + sid * 8,
                sid == NS - 1, EX2_OFF,
                scale=CH, offset=chunk)
            plsc.subcore_barrier()
            pltpu.sync_copy(agg_sh.at[pl.ds(row0, R_MAIN)],
                            out.at[pl.ds(row0, R_MAIN), pl.ds(chunk * CW, CW)])

            @pl.when(sid < R_EX_TILES)
            def _():
                pltpu.sync_copy(agg_sh.at[pl.ds(rex, 8)],
                                out.at[pl.ds(rex, 8), pl.ds(chunk * CW, CW)])

            plsc.subcore_barrier()

    @functools.partial(
        pl.kernel,
        out_type=jax.ShapeDtypeStruct((N, NC * K2), jnp.float32),
        mesh=mesh,
        compiler_params=params,
        scratch_types=[
            pltpu.VMEM_SHARED((N, K2), jnp.float32),
            pltpu.VMEM((ZROWS, K2), jnp.float32),
            pltpu.VMEM((G1, LANES), jnp.int32),
            pltpu.VMEM((G1, LANES), jnp.int32),
            pltpu.VMEM((SB, LANES, K2), jnp.float32),
            pltpu.SemaphoreType.DMA,
            pltpu.SemaphoreType.DMA,
        ],
    )
    def _sc2(t16, srcb, dstb, out, agg_sh, zb, si, di, msg, gsem, ssem):
        cid = lax.axis_index("c")
        sid = lax.axis_index("s")
        _zero_fill(zb, K2)
        row0 = sid * R_MAIN
        rex = NS * R_MAIN + sid * 8
        blk0 = cid * HB2 + sid * B_MAIN2
        _zero_rows(agg_sh, zb, ssem, row0, R_MAIN)

        @pl.when(sid < R_EX_TILES)
        def _():
            pltpu.sync_copy(zb.at[pl.ds(0, 8)], agg_sh.at[pl.ds(rex, 8)])

        plsc.subcore_barrier()
        _edge_pass(
            t16, lambda off, n: srcb.at[pl.ds(off, n)], dstb,
            agg_sh, si, di, msg, gsem, ssem,
            blk0, NG2, sid < 7 - cid, cid * HB2 + EX8_OFF2 + sid * 8,
            jnp.logical_and(cid == 1, sid == NS - 1), EX2_OFF)
        plsc.subcore_barrier()
        pltpu.sync_copy(agg_sh.at[pl.ds(row0, R_MAIN)],
                        out.at[pl.ds(row0, R_MAIN), pl.ds(cid * K2, K2)])

        @pl.when(sid < R_EX_TILES)
        def _():
            pltpu.sync_copy(agg_sh.at[pl.ds(rex, 8)],
                            out.at[pl.ds(rex, 8), pl.ds(cid * K2, K2)])

    return _sc1, _sc2


def _tcp_body(x_ref, lp_ref, xp_ref, xm_ref, m_ref):
    x = x_ref[...]
    m128 = jax.nn.sigmoid(lp_ref[...])
    xm_ref[...] = x * m128[:, :D]
    xp_ref[...] = jnp.concatenate(
        [x, jnp.ones((BN, 1), jnp.float32), jnp.zeros((BN, DP - D - 1),
                                                      jnp.float32)], axis=1)
    m_ref[...] = m128[:, :D]


def _tca_body(agg_ref, xp_ref, lp_ref, w1l_ref, w1r_ref, b1_ref,
              h_ref, ps_ref, psq_ref):
    m128 = jax.nn.sigmoid(lp_ref[...])
    aggc = agg_ref[...]
    cnt = jnp.maximum(aggc[:, D:D + 1], 1.0)
    mean_m = aggc * m128 / cnt
    xmm = xp_ref[...] * m128
    h = (jnp.dot(mean_m, w1l_ref[...], preferred_element_type=jnp.float32)
         + b1_ref[...]
         + jnp.dot(xmm, w1r_ref[...], preferred_element_type=jnp.float32))
    h_ref[...] = h
    ps_ref[...] = jnp.sum(h, axis=0, keepdims=True)[:, None, :]
    psq_ref[...] = jnp.sum(h * h, axis=0, keepdims=True)[:, None, :]


def _tcb_body(h_ref, ps_ref, psq_ref, g_ref, bb_ref, w2l_ref, w2r_ref, b2_ref,
              t_ref, u_ref):
    s = jnp.sum(ps_ref[...], axis=0)
    sq = jnp.sum(psq_ref[...], axis=0)
    mu = s * (1.0 / N)
    var = sq * (1.0 / N) - mu * mu
    inv = lax.rsqrt(var + 1e-5)
    hn = (h_ref[...] - mu) * (inv * g_ref[...]) + bb_ref[...]
    he = jnp.where(hn > 0, hn, jnp.exp(hn) - 1.0)
    ones_col = jnp.float32(1.0) * (jax.lax.broadcasted_iota(
        jnp.int32, (BN, K2), 1) == CNT2).astype(jnp.float32)
    t_ref[...] = (jnp.dot(he, w2l_ref[...], preferred_element_type=jnp.float32)
                  + ones_col)
    u_ref[...] = (jnp.dot(he, w2r_ref[...], preferred_element_type=jnp.float32)
                  + b2_ref[...])


def _tcc_body(p2_ref, u_ref, o_ref):
    p2 = p2_ref[...]
    ssum = p2[:, :K2] + p2[:, K2:]
    cnt = jnp.maximum(ssum[:, CNT2:CNT2 + 1], 1.0)
    o_ref[...] = ssum[:, :O] / cnt + u_ref[:, :O]


def kernel(x, edge_index, logits, W1l, b1, W1r, bn_gamma, bn_beta, W2l, b2, W2r):
    f32 = jnp.float32
    src = edge_index[0]
    dst = edge_index[1]

    lp = jnp.concatenate([logits, jnp.full((DP - D,), -1e30, f32)]).reshape(1, DP)

    xp, xm, m_out = pl.pallas_call(
        _tcp_body,
        grid=(GQ,),
        in_specs=[
            pl.BlockSpec((BN, D), lambda i: (i, 0)),
            pl.BlockSpec((1, DP), lambda i: (0, 0)),
        ],
        out_specs=[
            pl.BlockSpec((BN, DP), lambda i: (i, 0)),
            pl.BlockSpec((BN, D), lambda i: (i, 0)),
            pl.BlockSpec((1, D), lambda i: (0, 0)),
        ],
        out_shape=[
            jax.ShapeDtypeStruct((N, DP), f32),
            jax.ShapeDtypeStruct((N, D), f32),
            jax.ShapeDtypeStruct((1, D), f32),
        ],
    )(x, lp)

    xall = xp.reshape(CH * N, CW)
    srcb = src.reshape(EB, LANES)
    dstb = dst.reshape(EB, LANES)

    _sc1, _sc2 = _build_sc_kernels()
    agg = _sc1(xall, srcb, dstb)

    w1l_pad = jnp.concatenate([W1l, jnp.zeros((DP - D, H), f32)], axis=0)
    w1r_pad = jnp.concatenate([W1r, jnp.zeros((DP - D, H), f32)], axis=0)

    h_pre, ps, psq = pl.pallas_call(
        _tca_body,
        grid=(GQ,),
        in_specs=[
            pl.BlockSpec((BN, DP), lambda i: (i, 0)),
            pl.BlockSpec((BN, DP), lambda i: (i, 0)),
            pl.BlockSpec((1, DP), lambda i: (0, 0)),
            pl.BlockSpec((DP, H), lambda i: (0, 0)),
            pl.BlockSpec((DP, H), lambda i: (0, 0)),
            pl.BlockSpec((1, H), lambda i: (0, 0)),
        ],
        out_specs=[
            pl.BlockSpec((BN, H), lambda i: (i, 0)),
            pl.BlockSpec((1, 1, H), lambda i: (i, 0, 0)),
            pl.BlockSpec((1, 1, H), lambda i: (i, 0, 0)),
        ],
        out_shape=[
            jax.ShapeDtypeStruct((N, H), f32),
            jax.ShapeDtypeStruct((GQ, 1, H), f32),
            jax.ShapeDtypeStruct((GQ, 1, H), f32),
        ],
    )(agg, xp, lp, w1l_pad, w1r_pad, b1.reshape(1, H))

    w2l_pad = jnp.concatenate([W2l, jnp.zeros((H, K2 - O), f32)], axis=1)
    w2r_pad = jnp.concatenate([W2r, jnp.zeros((H, K2 - O), f32)], axis=1)
    b2_pad = jnp.concatenate([b2, jnp.zeros((K2 - O,), f32)]).reshape(1, K2)

    t16, u16 = pl.pallas_call(
        _tcb_body,
        grid=(GQ,),
        in_specs=[
            pl.BlockSpec((BN, H), lambda i: (i, 0)),
            pl.BlockSpec((GQ, 1, H), lambda i: (0, 0, 0)),
            pl.BlockSpec((GQ, 1, H), lambda i: (0, 0, 0)),
            pl.BlockSpec((1, H), lambda i: (0, 0)),
            pl.BlockSpec((1, H), lambda i: (0, 0)),
            pl.BlockSpec((H, K2), lambda i: (0, 0)),
            pl.BlockSpec((H, K2), lambda i: (0, 0)),
            pl.BlockSpec((1, K2), lambda i: (0, 0)),
        ],
        out_specs=[
            pl.BlockSpec((BN, K2), lambda i: (i, 0)),
            pl.BlockSpec((BN, K2), lambda i: (i, 0)),
        ],
        out_shape=[
            jax.ShapeDtypeStruct((N, K2), f32),
            jax.ShapeDtypeStruct((N, K2), f32),
        ],
    )(h_pre, ps, psq, bn_gamma.reshape(1, H), bn_beta.reshape(1, H),
      w2l_pad, w2r_pad, b2_pad)

    p2 = _sc2(t16, srcb, dstb)

    pred = pl.pallas_call(
        _tcc_body,
        grid=(GQ,),
        in_specs=[
            pl.BlockSpec((BN, NC * K2), lambda i: (i, 0)),
            pl.BlockSpec((BN, K2), lambda i: (i, 0)),
        ],
        out_specs=pl.BlockSpec((BN, O), lambda i: (i, 0)),
        out_shape=jax.ShapeDtypeStruct((N, O), f32),
    )(p2, u16)

    return (pred, xm, m_out.reshape(D))
